# Initial kernel scaffold; baseline (speedup 1.0000x reference)
import jax
import jax.numpy as jnp
from jax.experimental import pallas as pl


def kernel(x, edge_index, W1, b1, W2, b2):
    raise NotImplementedError("write your pallas kernel here")


# speedup vs baseline: 34.4449x; 34.4449x over previous
import functools

import jax
import jax.numpy as jnp
from jax import lax
from jax.experimental import pallas as pl
from jax.experimental.pallas import tpu as pltpu
from jax.experimental.pallas import tpu_sc as plsc

N = 10000
E = 320000
D_IN = 128
HID = 16
D_OUT = 64

NC = 2
NS = 16
NW = NC * NS
L = 16

EB = 125
RPW = 80
NPAD = 10240
NPT = NPAD // NS

_MESH = plsc.VectorSubcoreMesh(core_axis_name="c", subcore_axis_name="s")


def _sc_degree(dst3d):
    @functools.partial(
        pl.kernel,
        out_type=jax.ShapeDtypeStruct((NC, NPAD, HID), jnp.float32),
        mesh=_MESH,
        compiler_params=pltpu.CompilerParams(use_tc_tiling_on_sc=False),
        scratch_types=[
            pltpu.VMEM((EB, HID), jnp.float32),
            pltpu.VMEM((RPW, EB), jnp.int32),
            pltpu.VMEM((NPT, HID), jnp.float32),
            pltpu.VMEM_SHARED((NPAD, HID), jnp.float32),
        ],
    )
    def k(dst_hbm, deg_hbm, ones_v, didx_v, zbuf_v, deg_s):
        c = lax.axis_index("c")
        s = lax.axis_index("s")
        w = c * NS + s
        one = jnp.ones((L,), jnp.float32)
        zero = jnp.zeros((L,), jnp.float32)

        def fill(i, _):
            ones_v[i, :] = one
            zbuf_v[i, :] = zero
            return 0

        lax.fori_loop(0, EB, fill, 0)

        def zfill(i, _):
            zbuf_v[i, :] = zero
            return 0

        lax.fori_loop(EB, NPT, zfill, 0)
        pltpu.sync_copy(zbuf_v, deg_s.at[pl.ds(s * NPT, NPT)])
        pltpu.sync_copy(dst_hbm.at[w], didx_v)
        plsc.subcore_barrier()

        def body(j, _):
            pltpu.sync_copy(ones_v, deg_s.at[didx_v.at[j]], add=True)
            return 0

        lax.fori_loop(0, RPW, body, 0)
        plsc.subcore_barrier()
        pltpu.sync_copy(deg_s.at[pl.ds(s * NPT, NPT)],
                        deg_hbm.at[c].at[pl.ds(s * NPT, NPT)])

    return k(dst3d)


def _sc_aggregate(g, src3d, dst3d):
    @functools.partial(
        pl.kernel,
        out_type=jax.ShapeDtypeStruct((NC, NPAD, HID), jnp.float32),
        mesh=_MESH,
        compiler_params=pltpu.CompilerParams(use_tc_tiling_on_sc=False),
        scratch_types=[
            pltpu.VMEM((RPW, EB), jnp.int32),
            pltpu.VMEM((RPW, EB), jnp.int32),
            pltpu.VMEM((EB, HID), jnp.float32),
            pltpu.VMEM((NPT, HID), jnp.float32),
            pltpu.VMEM_SHARED((NPAD, HID), jnp.float32),
            pltpu.SemaphoreType.DMA,
        ],
    )
    def k(g_hbm, src_hbm, dst_hbm, out_hbm, sidx_v, didx_v, rows_v, zbuf_v,
          agg_s, sem):
        c = lax.axis_index("c")
        s = lax.axis_index("s")
        w = c * NS + s
        zero = jnp.zeros((L,), jnp.float32)

        def zfill(i, _):
            zbuf_v[i, :] = zero
            return 0

        lax.fori_loop(0, NPT, zfill, 0)
        pltpu.sync_copy(zbuf_v, agg_s.at[pl.ds(s * NPT, NPT)])
        pltpu.sync_copy(src_hbm.at[w], sidx_v)
        pltpu.sync_copy(dst_hbm.at[w], didx_v)
        plsc.subcore_barrier()

        def body(j, _):
            pltpu.async_copy(g_hbm.at[sidx_v.at[j]], rows_v, sem).wait()
            pltpu.sync_copy(rows_v, agg_s.at[didx_v.at[j]], add=True)
            return 0

        lax.fori_loop(0, RPW, body, 0)
        plsc.subcore_barrier()
        pltpu.sync_copy(agg_s.at[pl.ds(s * NPT, NPT)],
                        out_hbm.at[c].at[pl.ds(s * NPT, NPT)])

    return k(g, src3d, dst3d)


def _tc_layer1(x, W1, deg2):
    def body(x_ref, w_ref, deg_ref, g_ref, dinv_ref, sl_ref):
        h = jnp.dot(x_ref[...], w_ref[...], preferred_element_type=jnp.float32)
        deg = deg_ref[0, :N, :] + deg_ref[1, :N, :] + 1.0
        dinv = lax.rsqrt(deg)
        g_ref[...] = dinv * h
        dinv_ref[...] = dinv
        sl_ref[...] = dinv * dinv * h

    return pl.pallas_call(
        body,
        out_shape=[
            jax.ShapeDtypeStruct((N, HID), jnp.float32),
            jax.ShapeDtypeStruct((N, HID), jnp.float32),
            jax.ShapeDtypeStruct((N, HID), jnp.float32),
        ],
    )(x, W1, deg2)


def _tc_layer2_prep(agg1, dinv, sl1, b1):
    def body(agg_ref, dinv_ref, sl_ref, b_ref, g_ref, sl2_ref):
        dinv = dinv_ref[...]
        t = dinv * (agg_ref[0, :N, :] + agg_ref[1, :N, :]) + sl_ref[...] + b_ref[...]
        z = jnp.maximum(t, 0.0)
        g_ref[...] = dinv * z
        sl2_ref[...] = dinv * dinv * z

    return pl.pallas_call(
        body,
        out_shape=[
            jax.ShapeDtypeStruct((N, HID), jnp.float32),
            jax.ShapeDtypeStruct((N, HID), jnp.float32),
        ],
    )(agg1, dinv, sl1, b1)


def _tc_final(agg2, dinv, sl2, W2, b2):
    def body(agg_ref, dinv_ref, sl_ref, w_ref, b_ref, out_ref):
        u = dinv_ref[...] * (agg_ref[0, :N, :] + agg_ref[1, :N, :]) + sl_ref[...]
        o = jnp.dot(u, w_ref[...], preferred_element_type=jnp.float32) + b_ref[...]
        m = jnp.max(o, axis=1, keepdims=True)
        lse = m + jnp.log(jnp.sum(jnp.exp(o - m), axis=1, keepdims=True))
        out_ref[...] = o - lse

    return pl.pallas_call(
        body,
        out_shape=jax.ShapeDtypeStruct((N, D_OUT), jnp.float32),
    )(agg2, dinv, sl2, W2, b2)


def kernel(x, edge_index, W1, b1, W2, b2):
    src3d = edge_index[0].reshape(NW, RPW, EB)
    dst3d = edge_index[1].reshape(NW, RPW, EB)
    deg2 = _sc_degree(dst3d)
    g1, dinv, sl1 = _tc_layer1(x, W1, deg2)
    agg1 = _sc_aggregate(g1, src3d, dst3d)
    g2, sl2 = _tc_layer2_prep(agg1, dinv, sl1, b1.reshape(1, HID))
    agg2 = _sc_aggregate(g2, src3d, dst3d)
    return _tc_final(agg2, dinv, sl2, W2, b2.reshape(1, D_OUT))


# speedup vs baseline: 57.0896x; 1.6574x over previous
import functools

import jax
import jax.numpy as jnp
from jax import lax
from jax.experimental import pallas as pl
from jax.experimental.pallas import tpu as pltpu
from jax.experimental.pallas import tpu_sc as plsc

N = 10000
E = 320000
D_IN = 128
HID = 16
D_OUT = 64

NC = 2
NS = 16
NW = NC * NS
L = 16

EB = 125
RPW = 80
NBUF = 8
NGRP = RPW // NBUF
NPAD = 10240
NPT = NPAD // NS

_MESH = plsc.VectorSubcoreMesh(core_axis_name="c", subcore_axis_name="s")


def _sc_degree(dst3d):
    @functools.partial(
        pl.kernel,
        out_type=jax.ShapeDtypeStruct((NC, NPAD, HID), jnp.float32),
        mesh=_MESH,
        compiler_params=pltpu.CompilerParams(use_tc_tiling_on_sc=False),
        scratch_types=[
            pltpu.VMEM((EB, HID), jnp.float32),
            pltpu.VMEM((RPW, EB), jnp.int32),
            pltpu.VMEM((NPT, HID), jnp.float32),
            pltpu.VMEM_SHARED((NPAD, HID), jnp.float32),
            pltpu.SemaphoreType.DMA,
        ],
    )
    def k(dst_hbm, deg_hbm, ones_v, didx_v, zbuf_v, deg_s, sem):
        c = lax.axis_index("c")
        s = lax.axis_index("s")
        w = c * NS + s
        one = jnp.ones((L,), jnp.float32)
        zero = jnp.zeros((L,), jnp.float32)

        def fill(i, _):
            ones_v[i, :] = one
            zbuf_v[i, :] = zero
            return 0

        lax.fori_loop(0, EB, fill, 0)

        def zfill(i, _):
            zbuf_v[i, :] = zero
            return 0

        lax.fori_loop(EB, NPT, zfill, 0)
        pltpu.sync_copy(zbuf_v, deg_s.at[pl.ds(s * NPT, NPT)])
        pltpu.sync_copy(dst_hbm.at[w], didx_v)
        plsc.subcore_barrier()

        def body(j, _):
            pltpu.async_copy(ones_v, deg_s.at[didx_v.at[j]], sem, add=True)
            return 0

        lax.fori_loop(0, RPW, body, 0)

        def drain(j, _):
            pltpu.make_async_copy(ones_v, deg_s.at[didx_v.at[j]], sem).wait()
            return 0

        lax.fori_loop(0, RPW, drain, 0)
        plsc.subcore_barrier()
        pltpu.sync_copy(deg_s.at[pl.ds(s * NPT, NPT)],
                        deg_hbm.at[c].at[pl.ds(s * NPT, NPT)])

    return k(dst3d)


def _sc_aggregate(g, src3d, dst3d):
    @functools.partial(
        pl.kernel,
        out_type=jax.ShapeDtypeStruct((NC, NPAD, HID), jnp.float32),
        mesh=_MESH,
        compiler_params=pltpu.CompilerParams(use_tc_tiling_on_sc=False),
        scratch_types=[
            pltpu.VMEM((RPW, EB), jnp.int32),
            pltpu.VMEM((RPW, EB), jnp.int32),
            pltpu.VMEM((NBUF, EB, HID), jnp.float32),
            pltpu.VMEM((NPT, HID), jnp.float32),
            pltpu.VMEM_SHARED((NPAD, HID), jnp.float32),
            pltpu.SemaphoreType.DMA((NBUF,)),
            pltpu.SemaphoreType.DMA((NBUF,)),
        ],
    )
    def k(g_hbm, src_hbm, dst_hbm, out_hbm, sidx_v, didx_v, rows_v, zbuf_v,
          agg_s, gsem, ssem):
        c = lax.axis_index("c")
        s = lax.axis_index("s")
        w = c * NS + s
        zero = jnp.zeros((L,), jnp.float32)

        def zfill(i, _):
            zbuf_v[i, :] = zero
            return 0

        lax.fori_loop(0, NPT, zfill, 0)
        pltpu.sync_copy(zbuf_v, agg_s.at[pl.ds(s * NPT, NPT)])
        pltpu.sync_copy(src_hbm.at[w], sidx_v)
        pltpu.sync_copy(dst_hbm.at[w], didx_v)
        plsc.subcore_barrier()

        for b in range(NBUF):
            pltpu.async_copy(g_hbm.at[sidx_v.at[b]], rows_v.at[b], gsem.at[b])

        def group(g, _):
            sdescs = []
            for b in range(NBUF):
                j = g * NBUF + b
                pltpu.make_async_copy(g_hbm.at[sidx_v.at[j]], rows_v.at[b],
                                      gsem.at[b]).wait()
                sdescs.append(pltpu.async_copy(
                    rows_v.at[b], agg_s.at[didx_v.at[j]], ssem.at[b], add=True))
            for b in range(NBUF):
                sdescs[b].wait()

                @pl.when(g < NGRP - 1)
                def _(b=b):
                    jn = (g + 1) * NBUF + b
                    pltpu.async_copy(g_hbm.at[sidx_v.at[jn]], rows_v.at[b],
                                     gsem.at[b])

            return 0

        lax.fori_loop(0, NGRP, group, 0)
        plsc.subcore_barrier()
        pltpu.sync_copy(agg_s.at[pl.ds(s * NPT, NPT)],
                        out_hbm.at[c].at[pl.ds(s * NPT, NPT)])

    return k(g, src3d, dst3d)


def _tc_layer1(x, W1, deg2):
    def body(x_ref, w_ref, deg_ref, g_ref, dinv_ref, sl_ref):
        h = jnp.dot(x_ref[...], w_ref[...], preferred_element_type=jnp.float32)
        deg = deg_ref[0, :N, :] + deg_ref[1, :N, :] + 1.0
        dinv = lax.rsqrt(deg)
        g_ref[...] = dinv * h
        dinv_ref[...] = dinv
        sl_ref[...] = dinv * dinv * h

    return pl.pallas_call(
        body,
        out_shape=[
            jax.ShapeDtypeStruct((N, HID), jnp.float32),
            jax.ShapeDtypeStruct((N, HID), jnp.float32),
            jax.ShapeDtypeStruct((N, HID), jnp.float32),
        ],
    )(x, W1, deg2)


def _tc_layer2_prep(agg1, dinv, sl1, b1):
    def body(agg_ref, dinv_ref, sl_ref, b_ref, g_ref, sl2_ref):
        dinv = dinv_ref[...]
        t = dinv * (agg_ref[0, :N, :] + agg_ref[1, :N, :]) + sl_ref[...] + b_ref[...]
        z = jnp.maximum(t, 0.0)
        g_ref[...] = dinv * z
        sl2_ref[...] = dinv * dinv * z

    return pl.pallas_call(
        body,
        out_shape=[
            jax.ShapeDtypeStruct((N, HID), jnp.float32),
            jax.ShapeDtypeStruct((N, HID), jnp.float32),
        ],
    )(agg1, dinv, sl1, b1)


def _tc_final(agg2, dinv, sl2, W2, b2):
    def body(agg_ref, dinv_ref, sl_ref, w_ref, b_ref, out_ref):
        u = dinv_ref[...] * (agg_ref[0, :N, :] + agg_ref[1, :N, :]) + sl_ref[...]
        o = jnp.dot(u, w_ref[...], preferred_element_type=jnp.float32) + b_ref[...]
        m = jnp.max(o, axis=1, keepdims=True)
        lse = m + jnp.log(jnp.sum(jnp.exp(o - m), axis=1, keepdims=True))
        out_ref[...] = o - lse

    return pl.pallas_call(
        body,
        out_shape=jax.ShapeDtypeStruct((N, D_OUT), jnp.float32),
    )(agg2, dinv, sl2, W2, b2)


def kernel(x, edge_index, W1, b1, W2, b2):
    src3d = edge_index[0].reshape(NW, RPW, EB)
    dst3d = edge_index[1].reshape(NW, RPW, EB)
    deg2 = _sc_degree(dst3d)
    g1, dinv, sl1 = _tc_layer1(x, W1, deg2)
    agg1 = _sc_aggregate(g1, src3d, dst3d)
    g2, sl2 = _tc_layer2_prep(agg1, dinv, sl1, b1.reshape(1, HID))
    agg2 = _sc_aggregate(g2, src3d, dst3d)
    return _tc_final(agg2, dinv, sl2, W2, b2.reshape(1, D_OUT))


# speedup vs baseline: 64.8947x; 1.1367x over previous
import functools

import jax
import jax.numpy as jnp
from jax import lax
from jax.experimental import pallas as pl
from jax.experimental.pallas import tpu as pltpu
from jax.experimental.pallas import tpu_sc as plsc

N = 10000
E = 320000
D_IN = 128
HID = 16
D_OUT = 64

NC = 2
NS = 16
NW = NC * NS
L = 16

EB = 125
RPW = 80
NBUF = 8
NGRP = RPW // NBUF
NPAD = 10240
NPT = NPAD // NS

_MESH = plsc.VectorSubcoreMesh(core_axis_name="c", subcore_axis_name="s")


def _sc_degree(dst3d):
    @functools.partial(
        pl.kernel,
        out_type=jax.ShapeDtypeStruct((NC, NPAD, HID), jnp.float32),
        mesh=_MESH,
        compiler_params=pltpu.CompilerParams(use_tc_tiling_on_sc=False),
        scratch_types=[
            pltpu.VMEM((EB, HID), jnp.float32),
            pltpu.VMEM((RPW, EB), jnp.int32),
            pltpu.VMEM((NPT, HID), jnp.float32),
            pltpu.VMEM_SHARED((NPAD, HID), jnp.float32),
            pltpu.SemaphoreType.DMA,
        ],
    )
    def k(dst_hbm, deg_hbm, ones_v, didx_v, zbuf_v, deg_s, sem):
        c = lax.axis_index("c")
        s = lax.axis_index("s")
        w = c * NS + s
        one = jnp.ones((L,), jnp.float32)
        zero = jnp.zeros((L,), jnp.float32)

        def fill(i, _):
            ones_v[i, :] = one
            zbuf_v[i, :] = zero
            return 0

        lax.fori_loop(0, EB, fill, 0)

        def zfill(i, _):
            zbuf_v[i, :] = zero
            return 0

        lax.fori_loop(EB, NPT, zfill, 0)
        pltpu.sync_copy(zbuf_v, deg_s.at[pl.ds(s * NPT, NPT)])
        pltpu.sync_copy(dst_hbm.at[w], didx_v)
        plsc.subcore_barrier()

        def body(j, _):
            pltpu.async_copy(ones_v, deg_s.at[didx_v.at[j]], sem, add=True)
            return 0

        lax.fori_loop(0, RPW, body, 0)

        def drain(j, _):
            pltpu.make_async_copy(ones_v, deg_s.at[didx_v.at[j]], sem).wait()
            return 0

        lax.fori_loop(0, RPW, drain, 0)
        plsc.subcore_barrier()
        pltpu.sync_copy(deg_s.at[pl.ds(s * NPT, NPT)],
                        deg_hbm.at[c].at[pl.ds(s * NPT, NPT)])

    return k(dst3d)


def _sc_aggregate(g, src3d, dst3d):
    @functools.partial(
        pl.kernel,
        out_type=jax.ShapeDtypeStruct((NC, NPAD, HID), jnp.float32),
        mesh=_MESH,
        compiler_params=pltpu.CompilerParams(use_tc_tiling_on_sc=False),
        scratch_types=[
            pltpu.VMEM((RPW, EB), jnp.int32),
            pltpu.VMEM((RPW, EB), jnp.int32),
            pltpu.VMEM((NBUF, EB, HID), jnp.float32),
            pltpu.VMEM((NPT, HID), jnp.float32),
            pltpu.VMEM_SHARED((NPAD, HID), jnp.float32),
            pltpu.SemaphoreType.DMA((NBUF,)),
            pltpu.SemaphoreType.DMA((NBUF,)),
        ],
    )
    def k(g_hbm, src_hbm, dst_hbm, out_hbm, sidx_v, didx_v, rows_v, zbuf_v,
          agg_s, gsem, ssem):
        c = lax.axis_index("c")
        s = lax.axis_index("s")
        w = c * NS + s
        zero = jnp.zeros((L,), jnp.float32)

        def zfill(i, _):
            zbuf_v[i, :] = zero
            return 0

        lax.fori_loop(0, NPT, zfill, 0)
        pltpu.sync_copy(zbuf_v, agg_s.at[pl.ds(s * NPT, NPT)])
        pltpu.sync_copy(src_hbm.at[w], sidx_v)
        pltpu.sync_copy(dst_hbm.at[w], didx_v)
        plsc.subcore_barrier()

        for b in range(NBUF):
            pltpu.async_copy(g_hbm.at[sidx_v.at[b]], rows_v.at[b], gsem.at[b])

        def group(g, _):
            sdescs = []
            for b in range(NBUF):
                j = g * NBUF + b
                pltpu.make_async_copy(g_hbm.at[sidx_v.at[j]], rows_v.at[b],
                                      gsem.at[b]).wait()
                sdescs.append(pltpu.async_copy(
                    rows_v.at[b], agg_s.at[didx_v.at[j]], ssem.at[b], add=True))
            for b in range(NBUF):
                sdescs[b].wait()

                @pl.when(g < NGRP - 1)
                def _(b=b):
                    jn = (g + 1) * NBUF + b
                    pltpu.async_copy(g_hbm.at[sidx_v.at[jn]], rows_v.at[b],
                                     gsem.at[b])

            return 0

        lax.fori_loop(0, NGRP, group, 0)
        plsc.subcore_barrier()
        pltpu.sync_copy(agg_s.at[pl.ds(s * NPT, NPT)],
                        out_hbm.at[c].at[pl.ds(s * NPT, NPT)])

    return k(g, src3d, dst3d)


PK = 128 // HID
NPR = NPAD // PK


def _tc_matmul1(x, W1):
    def body(x_ref, w_ref, h_ref):
        h_ref[pl.ds(0, N), :] = jnp.dot(x_ref[...], w_ref[...],
                                        preferred_element_type=jnp.float32)
        h_ref[pl.ds(N, NPAD - N), :] = jnp.zeros((NPAD - N, HID), jnp.float32)

    return pl.pallas_call(
        body,
        out_shape=jax.ShapeDtypeStruct((NPAD, HID), jnp.float32),
    )(x, W1)


def _pack(h_ref):
    rowf = lax.broadcasted_iota(jnp.int32, (HID, 128), 0)
    colf = lax.broadcasted_iota(jnp.int32, (HID, 128), 1)
    hp = jnp.zeros((NPR, 128), jnp.float32)
    for j in range(PK):
        ej = (colf == rowf + HID * j).astype(jnp.float32)
        hp = hp + jnp.dot(h_ref[pl.ds(j * NPR, NPR), :], ej,
                          preferred_element_type=jnp.float32)
    return hp


def _tc_scale1(h1, deg2p):
    def body(h_ref, deg_ref, g_ref, dinv_ref, sl_ref):
        deg = deg_ref[0] + deg_ref[1] + 1.0
        dinv = lax.rsqrt(deg)
        hp = _pack(h_ref)
        g_ref[...] = dinv * hp
        dinv_ref[...] = dinv
        sl_ref[...] = dinv * dinv * hp

    return pl.pallas_call(
        body,
        out_shape=[
            jax.ShapeDtypeStruct((NPR, 128), jnp.float32),
            jax.ShapeDtypeStruct((NPR, 128), jnp.float32),
            jax.ShapeDtypeStruct((NPR, 128), jnp.float32),
        ],
    )(h1, deg2p)


def _tc_layer2_prep(agg1p, dinvp, sl1p, b1p):
    def body(agg_ref, dinv_ref, sl_ref, b_ref, g_ref, sl2_ref):
        dinv = dinv_ref[...]
        t = dinv * (agg_ref[0] + agg_ref[1]) + sl_ref[...] + b_ref[...]
        z = jnp.maximum(t, 0.0)
        g_ref[...] = dinv * z
        sl2_ref[...] = dinv * dinv * z

    return pl.pallas_call(
        body,
        out_shape=[
            jax.ShapeDtypeStruct((NPR, 128), jnp.float32),
            jax.ShapeDtypeStruct((NPR, 128), jnp.float32),
        ],
    )(agg1p, dinvp, sl1p, b1p)


def _tc_final(agg2p, dinvp, sl2p, W2t, b2):
    def body(agg_ref, dinv_ref, sl_ref, w_ref, b_ref, out_ref):
        up = dinv_ref[...] * (agg_ref[0] + agg_ref[1]) + sl_ref[...]
        blk = lax.broadcasted_iota(jnp.int32, (128, D_OUT), 0) // HID
        w = w_ref[...]
        for j in range(PK):
            mj = jnp.where(blk == j, w, 0.0)
            o = jnp.dot(up, mj, preferred_element_type=jnp.float32) + b_ref[...]
            m = jnp.max(o, axis=1, keepdims=True)
            lse = m + jnp.log(jnp.sum(jnp.exp(o - m), axis=1, keepdims=True))
            rows = min(NPR, N - j * NPR)
            out_ref[pl.ds(j * NPR, rows), :] = (o - lse)[:rows, :]

    return pl.pallas_call(
        body,
        out_shape=jax.ShapeDtypeStruct((N, D_OUT), jnp.float32),
    )(agg2p, dinvp, sl2p, W2t, b2)


def kernel(x, edge_index, W1, b1, W2, b2):
    psrc = (edge_index[0] % NPR) * PK + edge_index[0] // NPR
    pdst = (edge_index[1] % NPR) * PK + edge_index[1] // NPR
    src3d = psrc.reshape(NW, RPW, EB)
    dst3d = pdst.reshape(NW, RPW, EB)
    deg2 = _sc_degree(dst3d)
    h1 = _tc_matmul1(x, W1)
    g1p, dinvp, sl1p = _tc_scale1(h1, deg2.reshape(NC, NPR, 128))
    agg1 = _sc_aggregate(g1p.reshape(NPAD, HID), src3d, dst3d)
    b1p = jnp.tile(b1, PK).reshape(1, 128)
    g2p, sl2p = _tc_layer2_prep(agg1.reshape(NC, NPR, 128), dinvp, sl1p, b1p)
    agg2 = _sc_aggregate(g2p.reshape(NPAD, HID), src3d, dst3d)
    W2t = jnp.tile(W2, (PK, 1))
    return _tc_final(agg2.reshape(NC, NPR, 128), dinvp, sl2p, W2t,
                     b2.reshape(1, D_OUT))
